# baseline (speedup 1.0000x reference)
import functools

import jax
import jax.numpy as jnp
from jax import lax
from jax.experimental import pallas as pl
from jax.experimental.pallas import tpu as pltpu
from jax.experimental.pallas import tpu_sc as plsc

_B = 1024
_D = 64
_VOC = 1000
_VPAD = 1024
_TOK = 1000

_NC = 2
_NS = 16
_L = 16
_NW = _NC * _NS
_RPW = _B // _NW
_GRP = _RPW // _L
_GTOK = _L * _TOK
_GCNT = _L * _VPAD

_mesh = plsc.VectorSubcoreMesh(core_axis_name="c", subcore_axis_name="s")


@functools.partial(
    pl.kernel,
    mesh=_mesh,
    out_type=jax.ShapeDtypeStruct((_B * _VPAD,), jnp.float32),
    scratch_types=[
        pltpu.VMEM((_RPW * _TOK,), jnp.int32),
        pltpu.VMEM((_RPW * _VPAD,), jnp.float32),
        pltpu.SemaphoreType.DMA,
        pltpu.SemaphoreType.DMA,
        pltpu.SemaphoreType.DMA,
    ],
    compiler_params=pltpu.CompilerParams(needs_layout_passes=False),
)
def _hist(xc_hbm, counts_hbm, idx_v, counts_v, sem0, sem1, osem):
    wid = lax.axis_index("s") * _NC + lax.axis_index("c")
    ibase = wid * (_RPW * _TOK)
    obase = wid * (_RPW * _VPAD)
    in0 = pltpu.async_copy(
        xc_hbm.at[pl.ds(ibase, _GTOK)], idx_v.at[pl.ds(0, _GTOK)], sem0)
    in1 = pltpu.async_copy(
        xc_hbm.at[pl.ds(ibase + _GTOK, _GTOK)], idx_v.at[pl.ds(_GTOK, _GTOK)],
        sem1)

    lane = lax.iota(jnp.int32, _L)
    zeros = jnp.zeros((_L,), jnp.float32)
    ones = jnp.ones((_L,), jnp.float32)
    src0 = lane * _TOK
    src1 = src0 + _GTOK
    dst0 = lane * _VPAD
    dst1 = dst0 + _GCNT

    @plsc.parallel_loop(0, (_RPW * _VPAD) // _L, unroll=8)
    def _zero(i):
        counts_v[pl.ds(i * _L, _L)] = zeros

    in0.wait()

    @plsc.parallel_loop(0, _TOK, unroll=8)
    def _scat0(p):
        vals = plsc.load_gather(idx_v, [src0 + p])
        plsc.addupdate_scatter(counts_v, [dst0 + vals], ones)

    out0 = pltpu.async_copy(
        counts_v.at[pl.ds(0, _GCNT)], counts_hbm.at[pl.ds(obase, _GCNT)], osem)

    in1.wait()

    @plsc.parallel_loop(0, _TOK, unroll=8)
    def _scat1(p):
        vals = plsc.load_gather(idx_v, [src1 + p])
        plsc.addupdate_scatter(counts_v, [dst1 + vals], ones)

    pltpu.sync_copy(counts_v.at[pl.ds(_GCNT, _GCNT)],
                    counts_hbm.at[pl.ds(obase + _GCNT, _GCNT)])
    out0.wait()


def _mm(c_ref, e_ref, o_ref):
    o_ref[...] = lax.dot_general(
        c_ref[:, :_VOC], e_ref[...], (((1,), (0,)), ((), ())),
        preferred_element_type=jnp.float32,
    ) * (1.0 / _TOK)


_BM = 256


def kernel(word_pos, x, unused1, x_char, unused2, embedding_weight):
    xc_flat = x_char.reshape(-1)
    counts = _hist(xc_flat).reshape(_B, _VPAD)
    out = pl.pallas_call(
        _mm,
        grid=(_B // _BM,),
        in_specs=[
            pl.BlockSpec((_BM, _VPAD), lambda i: (i, 0)),
            pl.BlockSpec((_VOC, _D), lambda i: (0, 0)),
        ],
        out_specs=pl.BlockSpec((_BM, _D), lambda i: (i, 0)),
        out_shape=jax.ShapeDtypeStruct((_B, _D), jnp.float32),
    )(counts, embedding_weight)
    return out


# speedup vs baseline: 1.2442x; 1.2442x over previous
import functools

import jax
import jax.numpy as jnp
from jax import lax
from jax.experimental import pallas as pl
from jax.experimental.pallas import tpu as pltpu
from jax.experimental.pallas import tpu_sc as plsc

_B = 1024
_D = 64
_VOC = 1000
_VPAD = 1024
_W = 50
_C = 20
_TOK = _W * _C

_NC = 2
_NS = 16
_L = 16
_NW = _NC * _NS
_RPW = _B // _NW
_GRP = _RPW // _L

_mesh = plsc.VectorSubcoreMesh(core_axis_name="c", subcore_axis_name="s")


def _tr(x_ref, o_ref):
    o_ref[0] = x_ref[0].T


@functools.partial(
    pl.kernel,
    mesh=_mesh,
    out_type=jax.ShapeDtypeStruct((_B, _VPAD), jnp.float32),
    scratch_types=[
        pltpu.VMEM((_C, _RPW, _W), jnp.int32),
        pltpu.VMEM((_RPW, _VPAD), jnp.float32),
        pltpu.SemaphoreType.DMA,
    ],
    compiler_params=pltpu.CompilerParams(needs_layout_passes=False),
)
def _hist(xc_hbm, counts_hbm, idx_v, counts_v, sem):
    wid = lax.axis_index("s") * _NC + lax.axis_index("c")
    rbase = wid * _RPW
    in_dma = pltpu.async_copy(xc_hbm.at[:, pl.ds(rbase, _RPW), :], idx_v, sem)

    lane = lax.iota(jnp.int32, _L)
    zeros = jnp.zeros((_L,), jnp.float32)
    ones = jnp.ones((_L,), jnp.float32)
    rows = [lane + g * _L for g in range(_GRP)]

    @plsc.parallel_loop(0, (_RPW * _VPAD) // _L, unroll=8)
    def _zero(i):
        counts_v[i >> 6, pl.ds((i & 63) * _L, _L)] = zeros

    in_dma.wait()

    for c in range(_C):
        cvec = jnp.full((_L,), c, jnp.int32)

        @plsc.parallel_loop(0, _W, unroll=5)
        def _scat(w):
            wvec = jnp.broadcast_to(w, (_L,))
            for g in range(_GRP):
                vals = plsc.load_gather(idx_v, [cvec, rows[g], wvec])
                plsc.addupdate_scatter(counts_v, [rows[g], vals], ones)

    pltpu.sync_copy(counts_v, counts_hbm.at[pl.ds(rbase, _RPW)])


def _mmT(e_ref, c_ref, o_ref):
    o_ref[...] = lax.dot_general(
        e_ref[...], c_ref[:, :_VOC], (((1,), (1,)), ((), ())),
        preferred_element_type=jnp.float32,
    ) * (1.0 / _TOK)


_BM = 256


def kernel(word_pos, x, unused1, x_char, unused2, embedding_weight):
    xt = x_char.transpose(2, 1, 0)
    xcb = pl.pallas_call(
        _tr,
        grid=(_C,),
        in_specs=[pl.BlockSpec((1, _W, _B), lambda i: (i, 0, 0))],
        out_specs=pl.BlockSpec((1, _B, _W), lambda i: (i, 0, 0)),
        out_shape=jax.ShapeDtypeStruct((_C, _B, _W), jnp.int32),
    )(xt)
    counts = _hist(xcb)
    et = embedding_weight.T
    out_t = pl.pallas_call(
        _mmT,
        grid=(_B // _BM,),
        in_specs=[
            pl.BlockSpec((_D, _VOC), lambda i: (0, 0)),
            pl.BlockSpec((_BM, _VPAD), lambda i: (i, 0)),
        ],
        out_specs=pl.BlockSpec((_D, _BM), lambda i: (0, i)),
        out_shape=jax.ShapeDtypeStruct((_D, _B), jnp.float32),
    )(et, counts)
    return out_t.T


# speedup vs baseline: 1.4598x; 1.1733x over previous
import functools

import jax
import jax.numpy as jnp
from jax import lax
from jax.experimental import pallas as pl
from jax.experimental.pallas import tpu as pltpu
from jax.experimental.pallas import tpu_sc as plsc

_B = 1024
_D = 64
_VOC = 1000
_VPAD = 1024
_W = 50
_C = 20
_TOK = _W * _C

_NC = 2
_NS = 16
_L = 16
_NW = _NC * _NS
_RPW = _B // _NW
_GRP = _RPW // _L

_mesh = plsc.VectorSubcoreMesh(core_axis_name="c", subcore_axis_name="s")


def _tr(x_ref, i_ref, o_ref):
    ident = i_ref[...]
    for c in range(_C):
        xf = x_ref[c].astype(jnp.float32)
        t = lax.dot_general(xf, ident, (((0,), (0,)), ((), ())),
                            preferred_element_type=jnp.float32)
        o_ref[:, pl.ds(_W * c, _W)] = t.astype(jnp.int32)


@functools.partial(
    pl.kernel,
    mesh=_mesh,
    out_type=jax.ShapeDtypeStruct((_B, _VPAD), jnp.float32),
    scratch_types=[
        pltpu.VMEM((_RPW, _VOC), jnp.int32),
        pltpu.VMEM((_RPW, _VPAD), jnp.float32),
        pltpu.SemaphoreType.DMA,
    ],
    compiler_params=pltpu.CompilerParams(needs_layout_passes=False),
)
def _hist(xc_hbm, counts_hbm, idx_v, counts_v, sem):
    wid = lax.axis_index("s") * _NC + lax.axis_index("c")
    rbase = wid * _RPW
    in_dma = pltpu.async_copy(xc_hbm.at[pl.ds(rbase, _RPW)], idx_v, sem)

    lane = lax.iota(jnp.int32, _L)
    zeros = jnp.zeros((_L,), jnp.float32)
    ones = jnp.ones((_L,), jnp.float32)
    rows = [lane + g * _L for g in range(_GRP)]

    @plsc.parallel_loop(0, (_RPW * _VPAD) // _L, unroll=8)
    def _zero(i):
        counts_v[i >> 6, pl.ds((i & 63) * _L, _L)] = zeros

    in_dma.wait()

    @plsc.parallel_loop(0, _TOK, unroll=8)
    def _scat(p):
        pvec = jnp.broadcast_to(p, (_L,))
        for g in range(_GRP):
            vals = plsc.load_gather(idx_v, [rows[g], pvec])
            plsc.addupdate_scatter(counts_v, [rows[g], vals], ones)

    pltpu.sync_copy(counts_v, counts_hbm.at[pl.ds(rbase, _RPW)])


def _mmT(e_ref, c_ref, o_ref):
    o_ref[...] = lax.dot_general(
        e_ref[...], c_ref[:, :_VOC], (((1,), (1,)), ((), ())),
        preferred_element_type=jnp.float32,
    ) * (1.0 / _TOK)


_BM = 256


def kernel(word_pos, x, unused1, x_char, unused2, embedding_weight):
    xt = x_char.transpose(2, 1, 0)
    xcb = pl.pallas_call(
        _tr,
        in_specs=[
            pl.BlockSpec((_C, _W, _B), lambda: (0, 0, 0)),
            pl.BlockSpec((_W, _W), lambda: (0, 0)),
        ],
        out_specs=pl.BlockSpec((_B, _VOC), lambda: (0, 0)),
        out_shape=jax.ShapeDtypeStruct((_B, _VOC), jnp.int32),
    )(xt, jnp.eye(_W, dtype=jnp.float32))
    counts = _hist(xcb)
    et = embedding_weight.T
    out_t = pl.pallas_call(
        _mmT,
        grid=(_B // _BM,),
        in_specs=[
            pl.BlockSpec((_D, _VOC), lambda i: (0, 0)),
            pl.BlockSpec((_BM, _VPAD), lambda i: (i, 0)),
        ],
        out_specs=pl.BlockSpec((_D, _BM), lambda i: (0, i)),
        out_shape=jax.ShapeDtypeStruct((_D, _B), jnp.float32),
    )(et, counts)
    return out_t.T
